# baseline (speedup 1.0000x reference)
import functools

import jax
import jax.numpy as jnp
from jax import lax
from jax.experimental import pallas as pl
from jax.experimental.pallas import tpu as pltpu
from jax.experimental.pallas import tpu_sc as plsc

N = 10000
E = 320000
D = 128

NC = 2
NS = 16
NW = NC * NS
CHUNK = 96
NCH = 108
EP = NW * NCH * CHUNK
NP = 10112
RPS = NP // NS
NIDX = 4
NSTRIPE = RPS // CHUNK
RTAIL = RPS - NSTRIPE * CHUNK
PAD_DST = N


def _make_sc_aggregate(with_counts, nrows):
    mesh = plsc.VectorSubcoreMesh(core_axis_name="c", subcore_axis_name="s")
    la = nrows - 1

    out_type = [jax.ShapeDtypeStruct((NC * NP, D), jnp.bfloat16)]
    scratch = [
        [pltpu.VMEM((2, CHUNK), jnp.int32)] * NIDX,
        [pltpu.VMEM((CHUNK, D), jnp.bfloat16)] * nrows,
        pltpu.VMEM_SHARED((NP, D), jnp.bfloat16),
        [pltpu.SemaphoreType.DMA] * NIDX,
        [pltpu.SemaphoreType.DMA] * nrows,
        [pltpu.SemaphoreType.DMA] * nrows,
    ]
    if with_counts:
        out_type.append(jax.ShapeDtypeStruct((NC * NP, 16), jnp.float32))
        scratch += [
            pltpu.VMEM((CHUNK, 16), jnp.float32),
            pltpu.VMEM_SHARED((NP, 16), jnp.float32),
        ]

    @functools.partial(
        pl.kernel,
        out_type=out_type,
        mesh=mesh,
        compiler_params=pltpu.CompilerParams(use_tc_tiling_on_sc=False),
        scratch_types=scratch,
    )
    def k(*refs):
        if with_counts:
            (x_hbm, ei_hbm, zf_hbm, zc_hbm, ones_hbm,
             outf_hbm, outc_hbm, idxs, rows, acc, isem, gsem, ssem,
             onesv, cacc) = refs
        else:
            (x_hbm, ei_hbm, zf_hbm,
             outf_hbm, idxs, rows, acc, isem, gsem, ssem) = refs
        c = lax.axis_index("c")
        s = lax.axis_index("s")
        wid = s * NC + c

        def idx_load(m, q):
            return pltpu.make_async_copy(ei_hbm.at[wid * NCH + m], idxs[q],
                                         isem[q])

        def gather(m, b, q):
            return pltpu.make_async_copy(x_hbm.at[idxs[q].at[0]], rows[b],
                                         gsem[b])

        def scat(b, q):
            return pltpu.make_async_copy(rows[b], acc.at[idxs[q].at[1]],
                                         ssem[b])

        def cscat(b, q):
            return pltpu.make_async_copy(onesv, cacc.at[idxs[q].at[1]],
                                         ssem[b])

        pltpu.sync_copy(zf_hbm.at[pl.ds(0, CHUNK)], rows[0])
        for j in range(NSTRIPE):
            pltpu.sync_copy(rows[0], acc.at[pl.ds(s * RPS + j * CHUNK, CHUNK)])
        pltpu.sync_copy(rows[0].at[pl.ds(0, RTAIL)],
                        acc.at[pl.ds(s * RPS + NSTRIPE * CHUNK, RTAIL)])
        if with_counts:
            pltpu.sync_copy(zc_hbm.at[pl.ds(0, CHUNK)], onesv)
            for j in range(NSTRIPE):
                pltpu.sync_copy(onesv, cacc.at[pl.ds(s * RPS + j * CHUNK, CHUNK)])
            pltpu.sync_copy(onesv.at[pl.ds(0, RTAIL)],
                            cacc.at[pl.ds(s * RPS + NSTRIPE * CHUNK, RTAIL)])
            pltpu.sync_copy(ones_hbm, onesv)
        plsc.subcore_barrier()

        for m0 in range(la + 1):
            idx_load(m0, m0 % NIDX).start()
        for m0 in range(la):
            idx_load(m0, m0 % NIDX).wait()
            gather(m0, m0 % nrows, m0 % NIDX).start()

        UNROLL = nrows * NIDX // (2 if nrows % 2 == 0 else 1)
        assert UNROLL % nrows == 0 and UNROLL % NIDX == 0 and NCH % UNROLL == 0

        def body(g, carry):
            for u in range(UNROLL):
                m = g * UNROLL + u
                b, q = u % nrows, u % NIDX
                bd = (u - 1) % nrows
                qd = (u - 1) % NIDX
                qg = (u + la) % NIDX
                qp = (u + la + 1) % NIDX
                gather(m, b, q).wait()
                scat(b, q).start(add=True)
                if with_counts:
                    cscat(b, q).start(add=True)

                @pl.when(m > 0)
                def _():
                    scat(bd, qd).wait()
                    if with_counts:
                        cscat(bd, qd).wait()

                mn = lax.rem(m + la, NCH)
                idx_load(mn, qg).wait()
                gather(mn, bd, qg).start()
                mn2 = lax.rem(m + la + 1, NCH)
                idx_load(mn2, qp).start()
            return carry

        lax.fori_loop(0, NCH // UNROLL, body, 0)

        for w in range(la):
            mw = NCH - la + w
            gather(w, (mw - 1) % nrows, (NCH + w) % NIDX).wait()
        scat((NCH - 1) % nrows, (NCH - 1) % NIDX).wait()
        if with_counts:
            cscat((NCH - 1) % nrows, (NCH - 1) % NIDX).wait()
        idx_load(la, (NCH + la) % NIDX).wait()
        plsc.subcore_barrier()

        ob = c * NP + s * RPS
        for j in range(NSTRIPE):
            pltpu.sync_copy(acc.at[pl.ds(s * RPS + j * CHUNK, CHUNK)], rows[0])
            pltpu.sync_copy(rows[0], outf_hbm.at[pl.ds(ob + j * CHUNK, CHUNK)])
        pltpu.sync_copy(acc.at[pl.ds(s * RPS + NSTRIPE * CHUNK, RTAIL)],
                        rows[0].at[pl.ds(0, RTAIL)])
        pltpu.sync_copy(rows[0].at[pl.ds(0, RTAIL)],
                        outf_hbm.at[pl.ds(ob + NSTRIPE * CHUNK, RTAIL)])
        if with_counts:
            for j in range(NSTRIPE):
                pltpu.sync_copy(cacc.at[pl.ds(s * RPS + j * CHUNK, CHUNK)], onesv)
                pltpu.sync_copy(onesv, outc_hbm.at[pl.ds(ob + j * CHUNK, CHUNK)])
            pltpu.sync_copy(cacc.at[pl.ds(s * RPS + NSTRIPE * CHUNK, RTAIL)],
                            onesv.at[pl.ds(0, RTAIL)])
            pltpu.sync_copy(onesv.at[pl.ds(0, RTAIL)],
                            outc_hbm.at[pl.ds(ob + NSTRIPE * CHUNK, RTAIL)])

    return k


def _elu(v):
    return jnp.where(v > 0, v, jnp.exp(jnp.minimum(v, 0.0)) - 1.0)


def _softplus(v):
    return jnp.maximum(v, 0.0) + jnp.log(1.0 + jnp.exp(-jnp.abs(v)))


def _dot_t(a, w):
    return lax.dot_general(a, w, (((1,), (1,)), ((), ())),
                           preferred_element_type=jnp.float32)


TCB = 2000


def _tc_res_body(x, r1w, r1b, r2w, r2b, res1_o, res2_o):
    res1_o[...] = _dot_t(x[...], r1w[...]) + r1b[...]
    res2_o[...] = _dot_t(x[...], r2w[...]) + r2b[...]


def _tc1_body(x, pf, pc, wl1, bl1, wr1, h1_o, h1bf_o, inv_o):
    agg = pf[0].astype(jnp.float32) + pf[1].astype(jnp.float32)
    cnt = (pc[0] + pc[1])[:, 0:1]
    inv = 1.0 / jnp.maximum(cnt, 1.0)
    mean = agg * inv
    lin = _dot_t(mean, wl1[...]) + bl1[...] + _dot_t(x[...], wr1[...])
    h = _elu(lin)
    h1_o[...] = h
    h1bf_o[...] = h.astype(jnp.bfloat16)
    inv_o[...] = inv


def _tc2_body(h1, pf, inv, res1, res2, wl2, bl2, wr2,
              f1w, f1b, f2w, f2b, f3w, f3b, out_o):
    agg = pf[0].astype(jnp.float32) + pf[1].astype(jnp.float32)
    mean = agg * inv[...]
    h = _elu(_dot_t(mean, wl2[...]) + bl2[...] + _dot_t(h1[...], wr2[...]))
    h = h + res1[...]
    h = _elu(_dot_t(h, f1w[...]) + f1b[...])
    h = _elu(_dot_t(h, f2w[...]) + f2b[...]) + res2[...]
    out_o[...] = _softplus(_dot_t(h, f3w[...]) + f3b[...])


def _row_spec(width=D):
    return pl.BlockSpec((TCB, width), lambda i: (i, 0))


def _part_spec(width):
    return pl.BlockSpec((NC, TCB, width), lambda i: (0, i, 0))


def _w_spec():
    return pl.BlockSpec((D, D), lambda i: (0, 0))


def _b_spec():
    return pl.BlockSpec((1, D), lambda i: (0, 0))


def kernel(x, edge_index, conv1_Wl, conv1_bl, conv1_Wr, conv2_Wl, conv2_bl,
           conv2_Wr, res1_W, res1_b, res2_W, res2_b, fc1_W, fc1_b, fc2_W,
           fc2_b, fc3_W, fc3_b):
    ei = edge_index.astype(jnp.int32)
    npad = EP - E
    pad = jnp.stack([jnp.arange(npad, dtype=jnp.int32) % N,
                     PAD_DST + jnp.arange(npad, dtype=jnp.int32) % (NP - N)])
    ei_chunks = (jnp.concatenate([ei, pad], axis=1)
                 .reshape(2, NW * NCH, CHUNK).transpose(1, 0, 2))
    zf = jnp.zeros((CHUNK, D), jnp.bfloat16)
    zc = jnp.zeros((CHUNK, 16), jnp.float32)
    ones_c = jnp.ones((CHUNK, 16), jnp.float32)

    grid = (N // TCB,)
    res1, res2 = pl.pallas_call(
        _tc_res_body,
        grid=grid,
        in_specs=[_row_spec(), _w_spec(), _b_spec(), _w_spec(), _b_spec()],
        out_specs=[_row_spec(), _row_spec()],
        out_shape=[jax.ShapeDtypeStruct((N, D), jnp.float32)] * 2,
    )(x, res1_W, res1_b.reshape(1, D), res2_W, res2_b.reshape(1, D))

    x_bf = x.astype(jnp.bfloat16)
    pf1, pc1 = _make_sc_aggregate(True, 3)(x_bf, ei_chunks, zf, zc, ones_c)
    pf1 = pf1.reshape(NC, NP, D)
    pc1 = pc1.reshape(NC, NP, 16)

    h1, h1_bf, inv = pl.pallas_call(
        _tc1_body,
        grid=grid,
        in_specs=[
            _row_spec(), _part_spec(D), _part_spec(16),
            _w_spec(), _b_spec(), _w_spec(),
        ],
        out_specs=[_row_spec(), _row_spec(), _row_spec(1)],
        out_shape=[jax.ShapeDtypeStruct((N, D), jnp.float32),
                   jax.ShapeDtypeStruct((N, D), jnp.bfloat16),
                   jax.ShapeDtypeStruct((N, 1), jnp.float32)],
    )(x, pf1, pc1, conv1_Wl, conv1_bl.reshape(1, D), conv1_Wr)

    (pf2,) = _make_sc_aggregate(False, 3)(h1_bf, ei_chunks, zf)
    pf2 = pf2.reshape(NC, NP, D)

    out = pl.pallas_call(
        _tc2_body,
        grid=grid,
        in_specs=[
            _row_spec(), _part_spec(D), _row_spec(1), _row_spec(), _row_spec(),
            _w_spec(), _b_spec(), _w_spec(),
            _w_spec(), _b_spec(), _w_spec(), _b_spec(), _w_spec(), _b_spec(),
        ],
        out_specs=_row_spec(),
        out_shape=jax.ShapeDtypeStruct((N, D), jnp.float32),
    )(h1, pf2, inv, res1, res2,
      conv2_Wl, conv2_bl.reshape(1, D), conv2_Wr,
      fc1_W, fc1_b.reshape(1, D), fc2_W, fc2_b.reshape(1, D),
      fc3_W, fc3_b.reshape(1, D))

    return out


# speedup vs baseline: 1.0007x; 1.0007x over previous
import functools

import jax
import jax.numpy as jnp
from jax import lax
from jax.experimental import pallas as pl
from jax.experimental.pallas import tpu as pltpu
from jax.experimental.pallas import tpu_sc as plsc

N = 10000
E = 320000
D = 128

NC = 2
NS = 16
NW = NC * NS
CHUNK = 96
NCH = 108
EP = NW * NCH * CHUNK
NP = 10112
RPS = NP // NS
NIDX = 4
NSTRIPE = RPS // CHUNK
RTAIL = RPS - NSTRIPE * CHUNK
PAD_DST = N


def _make_sc_aggregate(with_counts, nrows):
    mesh = plsc.VectorSubcoreMesh(core_axis_name="c", subcore_axis_name="s")
    la = nrows - 1

    out_type = [jax.ShapeDtypeStruct((NC * NP, D), jnp.float32)]
    scratch = [
        [pltpu.VMEM((2, CHUNK), jnp.int32)] * NIDX,
        [pltpu.VMEM((CHUNK, D), jnp.float32)] * nrows,
        pltpu.VMEM_SHARED((NP, D), jnp.float32),
        [pltpu.SemaphoreType.DMA] * NIDX,
        [pltpu.SemaphoreType.DMA] * nrows,
        [pltpu.SemaphoreType.DMA] * nrows,
    ]
    if with_counts:
        out_type.append(jax.ShapeDtypeStruct((NC * NP, 16), jnp.float32))
        scratch += [
            pltpu.VMEM((CHUNK, 16), jnp.float32),
            pltpu.VMEM_SHARED((NP, 16), jnp.float32),
        ]

    @functools.partial(
        pl.kernel,
        out_type=out_type,
        mesh=mesh,
        compiler_params=pltpu.CompilerParams(use_tc_tiling_on_sc=False),
        scratch_types=scratch,
    )
    def k(*refs):
        if with_counts:
            (x_hbm, ei_hbm, zf_hbm, zc_hbm, ones_hbm,
             outf_hbm, outc_hbm, idxs, rows, acc, isem, gsem, ssem,
             onesv, cacc) = refs
        else:
            (x_hbm, ei_hbm, zf_hbm,
             outf_hbm, idxs, rows, acc, isem, gsem, ssem) = refs
        c = lax.axis_index("c")
        s = lax.axis_index("s")
        wid = s * NC + c

        def idx_load(m, q):
            return pltpu.make_async_copy(ei_hbm.at[wid * NCH + m], idxs[q],
                                         isem[q])

        def gather(m, b, q):
            return pltpu.make_async_copy(x_hbm.at[idxs[q].at[0]], rows[b],
                                         gsem[b])

        def scat(b, q):
            return pltpu.make_async_copy(rows[b], acc.at[idxs[q].at[1]],
                                         ssem[b])

        def cscat(b, q):
            return pltpu.make_async_copy(onesv, cacc.at[idxs[q].at[1]],
                                         ssem[b])

        pltpu.sync_copy(zf_hbm.at[pl.ds(0, CHUNK)], rows[0])
        for j in range(NSTRIPE):
            pltpu.sync_copy(rows[0], acc.at[pl.ds(s * RPS + j * CHUNK, CHUNK)])
        pltpu.sync_copy(rows[0].at[pl.ds(0, RTAIL)],
                        acc.at[pl.ds(s * RPS + NSTRIPE * CHUNK, RTAIL)])
        if with_counts:
            pltpu.sync_copy(zc_hbm.at[pl.ds(0, CHUNK)], onesv)
            for j in range(NSTRIPE):
                pltpu.sync_copy(onesv, cacc.at[pl.ds(s * RPS + j * CHUNK, CHUNK)])
            pltpu.sync_copy(onesv.at[pl.ds(0, RTAIL)],
                            cacc.at[pl.ds(s * RPS + NSTRIPE * CHUNK, RTAIL)])
            pltpu.sync_copy(ones_hbm, onesv)
        plsc.subcore_barrier()

        for m0 in range(la + 1):
            idx_load(m0, m0 % NIDX).start()
        for m0 in range(la):
            idx_load(m0, m0 % NIDX).wait()
            gather(m0, m0 % nrows, m0 % NIDX).start()

        UNROLL = nrows * NIDX // (2 if nrows % 2 == 0 else 1)
        assert UNROLL % nrows == 0 and UNROLL % NIDX == 0 and NCH % UNROLL == 0

        def body(g, carry):
            for u in range(UNROLL):
                m = g * UNROLL + u
                b, q = u % nrows, u % NIDX
                bd = (u - 1) % nrows
                qd = (u - 1) % NIDX
                qg = (u + la) % NIDX
                qp = (u + la + 1) % NIDX
                gather(m, b, q).wait()
                scat(b, q).start(add=True)
                if with_counts:
                    cscat(b, q).start(add=True)

                @pl.when(m > 0)
                def _():
                    scat(bd, qd).wait()
                    if with_counts:
                        cscat(bd, qd).wait()

                mn = lax.rem(m + la, NCH)
                idx_load(mn, qg).wait()
                gather(mn, bd, qg).start()
                mn2 = lax.rem(m + la + 1, NCH)
                idx_load(mn2, qp).start()
            return carry

        lax.fori_loop(0, NCH // UNROLL, body, 0)

        for w in range(la):
            mw = NCH - la + w
            gather(w, (mw - 1) % nrows, (NCH + w) % NIDX).wait()
        scat((NCH - 1) % nrows, (NCH - 1) % NIDX).wait()
        if with_counts:
            cscat((NCH - 1) % nrows, (NCH - 1) % NIDX).wait()
        idx_load(la, (NCH + la) % NIDX).wait()
        plsc.subcore_barrier()

        ob = c * NP + s * RPS
        for j in range(NSTRIPE):
            pltpu.sync_copy(acc.at[pl.ds(s * RPS + j * CHUNK, CHUNK)], rows[0])
            pltpu.sync_copy(rows[0], outf_hbm.at[pl.ds(ob + j * CHUNK, CHUNK)])
        pltpu.sync_copy(acc.at[pl.ds(s * RPS + NSTRIPE * CHUNK, RTAIL)],
                        rows[0].at[pl.ds(0, RTAIL)])
        pltpu.sync_copy(rows[0].at[pl.ds(0, RTAIL)],
                        outf_hbm.at[pl.ds(ob + NSTRIPE * CHUNK, RTAIL)])
        if with_counts:
            for j in range(NSTRIPE):
                pltpu.sync_copy(cacc.at[pl.ds(s * RPS + j * CHUNK, CHUNK)], onesv)
                pltpu.sync_copy(onesv, outc_hbm.at[pl.ds(ob + j * CHUNK, CHUNK)])
            pltpu.sync_copy(cacc.at[pl.ds(s * RPS + NSTRIPE * CHUNK, RTAIL)],
                            onesv.at[pl.ds(0, RTAIL)])
            pltpu.sync_copy(onesv.at[pl.ds(0, RTAIL)],
                            outc_hbm.at[pl.ds(ob + NSTRIPE * CHUNK, RTAIL)])

    return k


def _elu(v):
    return jnp.where(v > 0, v, jnp.exp(jnp.minimum(v, 0.0)) - 1.0)


def _softplus(v):
    return jnp.maximum(v, 0.0) + jnp.log(1.0 + jnp.exp(-jnp.abs(v)))


def _dot_t(a, w):
    return lax.dot_general(a, w, (((1,), (1,)), ((), ())),
                           preferred_element_type=jnp.float32)


TCB = 1000


def _tc_res_body(x, r1w, r1b, r2w, r2b, res1_o, res2_o):
    res1_o[...] = _dot_t(x[...], r1w[...]) + r1b[...]
    res2_o[...] = _dot_t(x[...], r2w[...]) + r2b[...]


def _tc1_body(x, pf, pc, wl1, bl1, wr1, h1_o, inv_o):
    agg = pf[0] + pf[1]
    cnt = (pc[0] + pc[1])[:, 0:1]
    inv = 1.0 / jnp.maximum(cnt, 1.0)
    mean = agg * inv
    lin = _dot_t(mean, wl1[...]) + bl1[...] + _dot_t(x[...], wr1[...])
    h1_o[...] = _elu(lin)
    inv_o[...] = inv


def _tc2_body(h1, pf, inv, res1, res2, wl2, bl2, wr2,
              f1w, f1b, f2w, f2b, f3w, f3b, out_o):
    agg = pf[0] + pf[1]
    mean = agg * inv[...]
    h = _elu(_dot_t(mean, wl2[...]) + bl2[...] + _dot_t(h1[...], wr2[...]))
    h = h + res1[...]
    h = _elu(_dot_t(h, f1w[...]) + f1b[...])
    h = _elu(_dot_t(h, f2w[...]) + f2b[...]) + res2[...]
    out_o[...] = _softplus(_dot_t(h, f3w[...]) + f3b[...])


def _row_spec(width=D):
    return pl.BlockSpec((TCB, width), lambda i: (i, 0))


def _part_spec(width):
    return pl.BlockSpec((NC, TCB, width), lambda i: (0, i, 0))


def _w_spec():
    return pl.BlockSpec((D, D), lambda i: (0, 0))


def _b_spec():
    return pl.BlockSpec((1, D), lambda i: (0, 0))


def kernel(x, edge_index, conv1_Wl, conv1_bl, conv1_Wr, conv2_Wl, conv2_bl,
           conv2_Wr, res1_W, res1_b, res2_W, res2_b, fc1_W, fc1_b, fc2_W,
           fc2_b, fc3_W, fc3_b):
    ei = edge_index.astype(jnp.int32)
    npad = EP - E
    pad = jnp.stack([jnp.arange(npad, dtype=jnp.int32) % N,
                     PAD_DST + jnp.arange(npad, dtype=jnp.int32) % (NP - N)])
    ei_chunks = (jnp.concatenate([ei, pad], axis=1)
                 .reshape(2, NW * NCH, CHUNK).transpose(1, 0, 2))
    zf = jnp.zeros((CHUNK, D), jnp.float32)
    zc = jnp.zeros((CHUNK, 16), jnp.float32)
    ones_c = jnp.ones((CHUNK, 16), jnp.float32)

    grid = (N // TCB,)
    res1, res2 = pl.pallas_call(
        _tc_res_body,
        grid=grid,
        in_specs=[_row_spec(), _w_spec(), _b_spec(), _w_spec(), _b_spec()],
        out_specs=[_row_spec(), _row_spec()],
        out_shape=[jax.ShapeDtypeStruct((N, D), jnp.float32)] * 2,
    )(x, res1_W, res1_b.reshape(1, D), res2_W, res2_b.reshape(1, D))

    pf1, pc1 = _make_sc_aggregate(True, 3)(x, ei_chunks, zf, zc, ones_c)
    pf1 = pf1.reshape(NC, NP, D)
    pc1 = pc1.reshape(NC, NP, 16)

    h1, inv = pl.pallas_call(
        _tc1_body,
        grid=grid,
        in_specs=[
            _row_spec(), _part_spec(D), _part_spec(16),
            _w_spec(), _b_spec(), _w_spec(),
        ],
        out_specs=[_row_spec(), _row_spec(1)],
        out_shape=[jax.ShapeDtypeStruct((N, D), jnp.float32),
                   jax.ShapeDtypeStruct((N, 1), jnp.float32)],
    )(x, pf1, pc1, conv1_Wl, conv1_bl.reshape(1, D), conv1_Wr)

    (pf2,) = _make_sc_aggregate(False, 3)(h1, ei_chunks, zf)
    pf2 = pf2.reshape(NC, NP, D)

    out = pl.pallas_call(
        _tc2_body,
        grid=grid,
        in_specs=[
            _row_spec(), _part_spec(D), _row_spec(1), _row_spec(), _row_spec(),
            _w_spec(), _b_spec(), _w_spec(),
            _w_spec(), _b_spec(), _w_spec(), _b_spec(), _w_spec(), _b_spec(),
        ],
        out_specs=_row_spec(),
        out_shape=jax.ShapeDtypeStruct((N, D), jnp.float32),
    )(h1, pf2, inv, res1, res2,
      conv2_Wl, conv2_bl.reshape(1, D), conv2_Wr,
      fc1_W, fc1_b.reshape(1, D), fc2_W, fc2_b.reshape(1, D),
      fc3_W, fc3_b.reshape(1, D))

    return out
